# baseline (speedup 1.0000x reference)
import functools

import jax
import jax.numpy as jnp
import numpy as np
from jax import lax
from jax.experimental import pallas as pl
from jax.experimental.pallas import tpu as pltpu
from jax.experimental.pallas import tpu_sc as plsc

TOP_K = 8
N_EXP = 64
N_GROUP = 8
GROUP_SIZE = 8
TOPK_GROUP = 4
SCALE = 2.5
HIDDEN = 4096

NEG_INF = np.float32(-np.inf)

_INFO = plsc.get_sparse_core_info()
NC, NS, L = _INFO.num_cores, _INFO.num_subcores, _INFO.num_lanes
NW = NC * NS


def _score_block(h_ref, wt_ref, out_ref):
    h = h_ref[...]
    wt = wt_ref[...]
    logits = jnp.dot(h, wt, preferred_element_type=jnp.float32)
    out_ref[0] = jax.nn.sigmoid(logits.T)


@functools.partial(jax.jit, static_argnames=("tb",))
def _scores(hidden_states, weight_t, tb):
    n_tokens = hidden_states.shape[0]
    nblk = n_tokens // tb
    return pl.pallas_call(
        _score_block,
        grid=(nblk,),
        in_specs=[
            pl.BlockSpec((tb, HIDDEN), lambda i: (i, 0)),
            pl.BlockSpec((HIDDEN, N_EXP), lambda i: (0, 0)),
        ],
        out_specs=pl.BlockSpec((1, N_EXP, tb), lambda i: (i, 0, 0)),
        out_shape=jax.ShapeDtypeStruct((nblk, N_EXP, tb), jnp.float32),
    )(hidden_states, weight_t)


def _splat_i(v):
    return jnp.full((L,), v, dtype=jnp.int32)


def _splat_f(v):
    return jnp.full((L,), v, dtype=jnp.float32)


def _make_router(tpw, n_iters):
    mesh = plsc.VectorSubcoreMesh(core_axis_name="c", subcore_axis_name="s")

    @functools.partial(
        pl.kernel,
        mesh=mesh,
        out_type=[
            jax.ShapeDtypeStruct((NW, TOP_K, tpw), jnp.int32),
            jax.ShapeDtypeStruct((NW, TOP_K, tpw), jnp.float32),
        ],
        compiler_params=pltpu.CompilerParams(use_tc_tiling_on_sc=False),
        scratch_types=[
            pltpu.VMEM((N_EXP, tpw), jnp.float32),
            pltpu.VMEM((TOP_K, tpw), jnp.int32),
            pltpu.VMEM((TOP_K, tpw), jnp.float32),
        ],
    )
    def router(scores_hbm, idx_hbm, w_hbm, sc_v, idx_v, w_v):
        wid = lax.axis_index("s") * NC + lax.axis_index("c")
        pltpu.sync_copy(scores_hbm.at[wid], sc_v)

        def body(j, _):
            col0 = j * L
            gs = []
            for g in range(N_GROUP):
                m1 = sc_v[g * GROUP_SIZE, pl.ds(col0, L)]
                m2 = _splat_f(NEG_INF)
                for e in range(1, GROUP_SIZE):
                    v = sc_v[g * GROUP_SIZE + e, pl.ds(col0, L)]
                    m2 = jnp.maximum(m2, jnp.minimum(v, m1))
                    m1 = jnp.maximum(m1, v)
                gs.append(m1 + m2)

            picks = []
            for _ in range(TOPK_GROUP):
                m = gs[0]
                for g in range(1, N_GROUP):
                    m = jnp.maximum(m, gs[g])
                pos = _splat_i(N_GROUP)
                for g in range(N_GROUP - 1, -1, -1):
                    pos = jnp.where(gs[g] == m, _splat_i(g), pos)
                picks.append(pos)
                for g in range(N_GROUP):
                    gs[g] = jnp.where(pos == _splat_i(g), _splat_f(NEG_INF), gs[g])

            def cas(a, b):
                return jnp.minimum(a, b), jnp.maximum(a, b)

            p0, p1, p2, p3 = picks
            p0, p1 = cas(p0, p1)
            p2, p3 = cas(p2, p3)
            p0, p2 = cas(p0, p2)
            p1, p3 = cas(p1, p3)
            p1, p2 = cas(p1, p2)
            rb = [p0 * _splat_i(GROUP_SIZE), p1 * _splat_i(GROUP_SIZE), p2 * _splat_i(GROUP_SIZE), p3 * _splat_i(GROUP_SIZE)]

            gsel = [p0, p1, p2, p3]
            cs = [_splat_f(0.0) for _ in range(TOPK_GROUP * GROUP_SIZE)]
            for g in range(N_GROUP):
                hits = [gsel[s] == _splat_i(g) for s in range(TOPK_GROUP)]
                for e in range(GROUP_SIZE):
                    v = sc_v[g * GROUP_SIZE + e, pl.ds(col0, L)]
                    for s in range(TOPK_GROUP):
                        cs[s * GROUP_SIZE + e] = jnp.where(
                            hits[s], v, cs[s * GROUP_SIZE + e])
            ncand = len(cs)

            vals = []
            for k in range(TOP_K):
                m = cs[0]
                for i in range(1, ncand):
                    m = jnp.maximum(m, cs[i])
                pos = _splat_i(ncand)
                ei = _splat_i(0)
                for i in range(ncand - 1, -1, -1):
                    eq = cs[i] == m
                    pos = jnp.where(eq, _splat_i(i), pos)
                    ei = jnp.where(eq, rb[i // GROUP_SIZE] + _splat_i(i % GROUP_SIZE), ei)
                idx_v[k, pl.ds(col0, L)] = ei
                vals.append(m)
                for i in range(ncand):
                    cs[i] = jnp.where(pos == _splat_i(i), _splat_f(-1.0), cs[i])

            denom = vals[0]
            for k in range(1, TOP_K):
                denom = denom + vals[k]
            inv = _splat_f(SCALE) / (denom + _splat_f(1e-20))
            for k in range(TOP_K):
                w_v[k, pl.ds(col0, L)] = vals[k] * inv
            return 0

        lax.fori_loop(0, n_iters, body, 0)
        pltpu.sync_copy(idx_v, idx_hbm.at[wid])
        pltpu.sync_copy(w_v, w_hbm.at[wid])

    return router


N_CHUNKS = 2


def kernel(hidden_states, weight, e_score_correction_bias):
    hidden_states = hidden_states.reshape(-1, HIDDEN).astype(jnp.float32)
    n_tokens = hidden_states.shape[0]
    weight_t = weight.astype(jnp.float32).T
    ctok = n_tokens // N_CHUNKS
    tpw = ctok // NW
    router = _make_router(tpw, tpw // L)
    idxs, ws = [], []
    for c in range(N_CHUNKS):
        h_c = hidden_states[c * ctok:(c + 1) * ctok]
        scores3 = _scores(h_c, weight_t, tpw)
        idx3, w3 = router(scores3)
        idxs.append(idx3.transpose(0, 2, 1).reshape(ctok, TOP_K))
        ws.append(w3.transpose(0, 2, 1).reshape(ctok, TOP_K))
    return (jnp.concatenate(idxs, axis=0), jnp.concatenate(ws, axis=0))


# speedup vs baseline: 2.4647x; 2.4647x over previous
import functools

import jax
import jax.numpy as jnp
import numpy as np
from jax import lax
from jax.experimental import pallas as pl
from jax.experimental.pallas import tpu as pltpu
from jax.experimental.pallas import tpu_sc as plsc

TOP_K = 8
N_EXP = 64
N_GROUP = 8
GROUP_SIZE = 8
TOPK_GROUP = 4
SCALE = 2.5
HIDDEN = 4096

NEG_INF = np.float32(-np.inf)

_INFO = plsc.get_sparse_core_info()
NC, NS, L = _INFO.num_cores, _INFO.num_subcores, _INFO.num_lanes
NW = NC * NS


def _score_block(h_ref, wt_ref, out_ref):
    h = h_ref[...]
    wt = wt_ref[...]
    logits = jnp.dot(h, wt, preferred_element_type=jnp.float32)
    out_ref[0] = jax.nn.sigmoid(logits.T)


@functools.partial(jax.jit, static_argnames=("tb", "nblk", "blk0"))
def _scores(hidden_states, weight_t, tb, nblk, blk0):
    return pl.pallas_call(
        _score_block,
        grid=(nblk,),
        in_specs=[
            pl.BlockSpec((tb, HIDDEN), lambda i: (blk0 + i, 0)),
            pl.BlockSpec((HIDDEN, N_EXP), lambda i: (0, 0)),
        ],
        out_specs=pl.BlockSpec((1, N_EXP, tb), lambda i: (i, 0, 0)),
        out_shape=jax.ShapeDtypeStruct((nblk, N_EXP, tb), jnp.float32),
    )(hidden_states, weight_t)


def _splat_i(v):
    return jnp.full((L,), v, dtype=jnp.int32)


def _splat_f(v):
    return jnp.full((L,), v, dtype=jnp.float32)


def _make_router(tpw, n_iters):
    mesh = plsc.VectorSubcoreMesh(core_axis_name="c", subcore_axis_name="s")

    @functools.partial(
        pl.kernel,
        mesh=mesh,
        out_type=[
            jax.ShapeDtypeStruct((NW, TOP_K, tpw), jnp.int32),
            jax.ShapeDtypeStruct((NW, TOP_K, tpw), jnp.float32),
        ],
        compiler_params=pltpu.CompilerParams(use_tc_tiling_on_sc=False),
        scratch_types=[
            pltpu.VMEM((N_EXP, tpw), jnp.float32),
            pltpu.VMEM((TOP_K, tpw), jnp.int32),
            pltpu.VMEM((TOP_K, tpw), jnp.float32),
        ],
    )
    def router(scores_hbm, idx_hbm, w_hbm, sc_v, idx_v, w_v):
        wid = lax.axis_index("s") * NC + lax.axis_index("c")
        pltpu.sync_copy(scores_hbm.at[wid], sc_v)

        def body(j, _):
            col0 = j * L
            gs = []
            for g in range(N_GROUP):
                m1 = sc_v[g * GROUP_SIZE, pl.ds(col0, L)]
                m2 = _splat_f(NEG_INF)
                for e in range(1, GROUP_SIZE):
                    v = sc_v[g * GROUP_SIZE + e, pl.ds(col0, L)]
                    m2 = jnp.maximum(m2, jnp.minimum(v, m1))
                    m1 = jnp.maximum(m1, v)
                gs.append(m1 + m2)

            picks = []
            for _ in range(TOPK_GROUP):
                m = gs[0]
                for g in range(1, N_GROUP):
                    m = jnp.maximum(m, gs[g])
                pos = _splat_i(N_GROUP)
                for g in range(N_GROUP - 1, -1, -1):
                    pos = jnp.where(gs[g] == m, _splat_i(g), pos)
                picks.append(pos)
                for g in range(N_GROUP):
                    gs[g] = jnp.where(pos == _splat_i(g), _splat_f(NEG_INF), gs[g])

            def cas(a, b):
                return jnp.minimum(a, b), jnp.maximum(a, b)

            p0, p1, p2, p3 = picks
            p0, p1 = cas(p0, p1)
            p2, p3 = cas(p2, p3)
            p0, p2 = cas(p0, p2)
            p1, p3 = cas(p1, p3)
            p1, p2 = cas(p1, p2)
            rb = [p0 * _splat_i(GROUP_SIZE), p1 * _splat_i(GROUP_SIZE), p2 * _splat_i(GROUP_SIZE), p3 * _splat_i(GROUP_SIZE)]

            gsel = [p0, p1, p2, p3]
            cs = [_splat_f(0.0) for _ in range(TOPK_GROUP * GROUP_SIZE)]
            for g in range(N_GROUP):
                hits = [gsel[s] == _splat_i(g) for s in range(TOPK_GROUP)]
                for e in range(GROUP_SIZE):
                    v = sc_v[g * GROUP_SIZE + e, pl.ds(col0, L)]
                    for s in range(TOPK_GROUP):
                        cs[s * GROUP_SIZE + e] = jnp.where(
                            hits[s], v, cs[s * GROUP_SIZE + e])
            ncand = len(cs)

            vals = []
            for k in range(TOP_K):
                m = cs[0]
                for i in range(1, ncand):
                    m = jnp.maximum(m, cs[i])
                pos = _splat_i(ncand)
                ei = _splat_i(0)
                for i in range(ncand - 1, -1, -1):
                    eq = cs[i] == m
                    pos = jnp.where(eq, _splat_i(i), pos)
                    ei = jnp.where(eq, rb[i // GROUP_SIZE] + _splat_i(i % GROUP_SIZE), ei)
                idx_v[k, pl.ds(col0, L)] = ei
                vals.append(m)
                for i in range(ncand):
                    cs[i] = jnp.where(pos == _splat_i(i), _splat_f(-1.0), cs[i])

            denom = vals[0]
            for k in range(1, TOP_K):
                denom = denom + vals[k]
            inv = _splat_f(SCALE) / (denom + _splat_f(1e-20))
            for k in range(TOP_K):
                w_v[k, pl.ds(col0, L)] = vals[k] * inv
            return 0

        lax.fori_loop(0, n_iters, body, 0)
        pltpu.sync_copy(idx_v, idx_hbm.at[wid])
        pltpu.sync_copy(w_v, w_hbm.at[wid])

    return router


N_CHUNKS = 2


def kernel(hidden_states, weight, e_score_correction_bias):
    hidden_states = hidden_states.reshape(-1, HIDDEN).astype(jnp.float32)
    n_tokens = hidden_states.shape[0]
    weight_t = weight.astype(jnp.float32).T
    ctok = n_tokens // N_CHUNKS
    tpw = ctok // NW
    router = _make_router(tpw, tpw // L)
    idxs, ws = [], []
    for c in range(N_CHUNKS):
        scores3 = _scores(hidden_states, weight_t, tpw, NW, c * NW)
        idx3, w3 = router(scores3)
        idxs.append(idx3.transpose(0, 2, 1).reshape(ctok, TOP_K))
        ws.append(w3.transpose(0, 2, 1).reshape(ctok, TOP_K))
    return (jnp.concatenate(idxs, axis=0), jnp.concatenate(ws, axis=0))
